# baseline (speedup 1.0000x reference)
import jax
import jax.numpy as jnp
from jax import lax
from jax.experimental import pallas as pl
from jax.experimental.pallas import tpu as pltpu
from jax.experimental.pallas import tpu_sc as plsc

BATCH = 16384
DIM = 64
NW = 32
B_PER_W = BATCH // NW
N_CHUNK = 4
CHUNK = B_PER_W // N_CHUNK


def _sgns_body(t_ids_hbm, c_ids_hbm, t_w_hbm, c_w_hbm, out_hbm,
               idx_t_v, idx_c_v, rows_t_v, rows_c_v, part_v, out_v, sem):
    wid = lax.axis_index("s") * 2 + lax.axis_index("c")
    base = wid * B_PER_W

    pltpu.sync_copy(t_ids_hbm.at[pl.ds(wid * N_CHUNK, N_CHUNK)], idx_t_v)
    pltpu.sync_copy(c_ids_hbm.at[pl.ds(wid * N_CHUNK, N_CHUNK)], idx_c_v)

    copies = []
    for j in range(N_CHUNK):
        copies.append(pltpu.async_copy(
            t_w_hbm.at[idx_t_v.at[j]], rows_t_v.at[pl.ds(j * CHUNK, CHUNK)], sem))
        copies.append(pltpu.async_copy(
            c_w_hbm.at[idx_c_v.at[j]], rows_c_v.at[pl.ds(j * CHUNK, CHUNK)], sem))
    for c in copies:
        c.wait()

    lane17 = lax.iota(jnp.int32, 16) * 17

    def group(g, _):
        for i in range(16):
            r = g * 16 + i
            acc = rows_t_v[r, pl.ds(0, 16)] * rows_c_v[r, pl.ds(0, 16)]
            for k in range(1, 4):
                acc = acc + (rows_t_v[r, pl.ds(k * 16, 16)]
                             * rows_c_v[r, pl.ds(k * 16, 16)])
            plsc.store_scatter(part_v, [lane17 + i], acc)
        s = part_v[pl.ds(0, 16)]
        for l in range(1, 16):
            s = s + part_v[pl.ds(l * 17, 16)]
        out_v[pl.ds(g * 16, 16)] = s
        return 0

    lax.fori_loop(0, B_PER_W // 16, group, 0)

    pltpu.sync_copy(out_v, out_hbm.at[pl.ds(base, B_PER_W)])


@jax.jit
def kernel(t_ids, c_ids, t_in_w, c_out_w):
    t_ids2 = jnp.asarray(t_ids, jnp.int32).reshape(NW * N_CHUNK, CHUNK)
    c_ids2 = jnp.asarray(c_ids, jnp.int32).reshape(NW * N_CHUNK, CHUNK)

    mesh = plsc.VectorSubcoreMesh(core_axis_name="c", subcore_axis_name="s")
    f = pl.kernel(
        _sgns_body,
        out_type=jax.ShapeDtypeStruct((BATCH,), jnp.float32),
        mesh=mesh,
        compiler_params=pltpu.CompilerParams(
            needs_layout_passes=False, use_tc_tiling_on_sc=False),
        scratch_types=[
            pltpu.VMEM((N_CHUNK, CHUNK), jnp.int32),
            pltpu.VMEM((N_CHUNK, CHUNK), jnp.int32),
            pltpu.VMEM((B_PER_W, DIM), jnp.float32),
            pltpu.VMEM((B_PER_W, DIM), jnp.float32),
            pltpu.VMEM((16 * 17,), jnp.float32),
            pltpu.VMEM((B_PER_W,), jnp.float32),
            pltpu.SemaphoreType.DMA,
        ],
    )
    return f(t_ids2, c_ids2, t_in_w, c_out_w)


# speedup vs baseline: 2.1955x; 2.1955x over previous
import jax
import jax.numpy as jnp
from jax import lax
from jax.experimental import pallas as pl
from jax.experimental.pallas import tpu as pltpu
from jax.experimental.pallas import tpu_sc as plsc

N_ROWS = 1000000
DIM = 64
BATCH = 16384
NW = 32
B_PER_W = BATCH // NW
C = 16
NCH = B_PER_W // C


def _sgns_body(t_ids_hbm, c_ids_hbm, t_w_hbm, c_w_hbm, out_hbm,
               idx_t_v, idx_c_v,
               bt0, bt1, bc0, bc1, part_v, out_v, sem0, sem1):
    wid = lax.axis_index("s") * 2 + lax.axis_index("c")

    pltpu.sync_copy(t_ids_hbm.at[wid], idx_t_v)
    pltpu.sync_copy(c_ids_hbm.at[wid], idx_c_v)

    bufs = ((bt0, bc0, sem0), (bt1, bc1, sem1))

    def fire(j, par):
        bt, bc, sem = bufs[par]
        tqv = lax.shift_right_logical(idx_t_v[pl.ds(j * C, C)], 3)
        cqv = lax.shift_right_logical(idx_c_v[pl.ds(j * C, C)], 3)
        for i in range(C):
            pltpu.async_copy(t_w_hbm.at[tqv[i]], bt.at[i], sem)
            pltpu.async_copy(c_w_hbm.at[cqv[i]], bc.at[i], sem)

    def drain(par):
        bt, bc, sem = bufs[par]
        pltpu.make_async_copy(t_w_hbm.at[pl.ds(0, C)], bt, sem).wait()
        pltpu.make_async_copy(c_w_hbm.at[pl.ds(0, C)], bc, sem).wait()

    lane17 = lax.iota(jnp.int32, 16) * 17

    def compute(j, par):
        bt, bc, _ = bufs[par]
        stv = lax.bitwise_and(idx_t_v[pl.ds(j * C, C)], 7)
        scv = lax.bitwise_and(idx_c_v[pl.ds(j * C, C)], 7)
        for i in range(C):
            st = stv[i]
            sc = scv[i]
            acc = bt[i, st, pl.ds(0, 16)] * bc[i, sc, pl.ds(0, 16)]
            for k in range(1, 4):
                acc = acc + (bt[i, st, pl.ds(k * 16, 16)]
                             * bc[i, sc, pl.ds(k * 16, 16)])
            plsc.store_scatter(part_v, [lane17 + i], acc)
        s = part_v[pl.ds(0, 16)]
        for l in range(1, 16):
            s = s + part_v[pl.ds(l * 17, 16)]
        out_v[pl.ds(j * C, 16)] = s

    fire(0, 0)
    fire(1, 1)

    def body(g, _):
        j0 = g * 2
        drain(0)
        compute(j0, 0)

        @pl.when(g < NCH // 2 - 1)
        def _():
            fire(j0 + 2, 0)

        drain(1)
        compute(j0 + 1, 1)

        @pl.when(g < NCH // 2 - 1)
        def _():
            fire(j0 + 3, 1)

        return 0

    lax.fori_loop(0, NCH // 2, body, 0)

    pltpu.sync_copy(out_v, out_hbm.at[pl.ds(wid * B_PER_W, B_PER_W)])


@jax.jit
def kernel(t_ids, c_ids, t_in_w, c_out_w):
    t_ids2 = jnp.asarray(t_ids, jnp.int32).reshape(NW, B_PER_W)
    c_ids2 = jnp.asarray(c_ids, jnp.int32).reshape(NW, B_PER_W)
    t_w3 = t_in_w.reshape(N_ROWS // 8, 8, DIM)
    c_w3 = c_out_w.reshape(N_ROWS // 8, 8, DIM)

    mesh = plsc.VectorSubcoreMesh(core_axis_name="c", subcore_axis_name="s")
    f = pl.kernel(
        _sgns_body,
        out_type=jax.ShapeDtypeStruct((BATCH,), jnp.float32),
        mesh=mesh,
        compiler_params=pltpu.CompilerParams(needs_layout_passes=False),
        scratch_types=[
            pltpu.VMEM((B_PER_W,), jnp.int32),
            pltpu.VMEM((B_PER_W,), jnp.int32),
            pltpu.VMEM((C, 8, DIM), jnp.float32),
            pltpu.VMEM((C, 8, DIM), jnp.float32),
            pltpu.VMEM((C, 8, DIM), jnp.float32),
            pltpu.VMEM((C, 8, DIM), jnp.float32),
            pltpu.VMEM((16 * 17,), jnp.float32),
            pltpu.VMEM((B_PER_W,), jnp.float32),
            pltpu.SemaphoreType.DMA,
            pltpu.SemaphoreType.DMA,
        ],
    )
    return f(t_ids2, c_ids2, t_w3, c_w3)
